# baseline (device time: 133112 ns/iter reference)
import jax
import jax.numpy as jnp
from jax import lax
from jax.experimental import pallas as pl
from jax.experimental.pallas import tpu as pltpu

N_DEV = 8
M_PER = 512
N_HOP = 3


def kernel(x, w_mat, scale_x, scale_w):
    m_per, k = x.shape
    _, n_per = w_mat.shape
    scale = (scale_x[0] * scale_w[0]).reshape(1, 1)

    def body(x_ref, w_ref, scale_ref, out_ref,
             allx_ref, fwd_send, fwd_recv, bwd_send, bwd_recv,
             z_send, z_recv):
        my = lax.axis_index("i")
        right = (my + 1) % N_DEV
        left = (my + N_DEV - 1) % N_DEV
        anti = (my + 4) % N_DEV

        barrier = pltpu.get_barrier_semaphore()
        for nbr in (left, right, anti):
            pl.semaphore_signal(barrier, inc=1, device_id=(nbr,),
                                device_id_type=pl.DeviceIdType.MESH)
        pl.semaphore_wait(barrier, 3)

        def gemm(chunk, row0):
            acc = lax.dot_general(
                chunk.astype(jnp.bfloat16), w_ref[...].astype(jnp.bfloat16),
                dimension_numbers=(((1,), (0,)), ((), ())),
                preferred_element_type=jnp.float32,
            )
            out_ref[pl.ds(row0, chunk.shape[0]), :] = acc * scale_ref[0, 0]

        def rdma(origin, send_sems, recv_sems, h, dst):
            return pltpu.make_async_remote_copy(
                src_ref=allx_ref.at[origin], dst_ref=allx_ref.at[origin],
                send_sem=send_sems.at[h], recv_sem=recv_sems.at[h],
                device_id=(dst,), device_id_type=pl.DeviceIdType.MESH,
            )

        allx_ref[my] = x_ref[...]

        rdma(my, z_send, z_recv, 0, anti).start()
        rdma(my, fwd_send, fwd_recv, 0, right).start()
        rdma(my, bwd_send, bwd_recv, 0, left).start()

        gemm(x_ref[...], my * M_PER)

        for h in range(N_HOP):
            of_r = (my + N_DEV - h - 1) % N_DEV
            ob_r = (my + h + 1) % N_DEV

            rdma(of_r, fwd_send, fwd_recv, h, right).wait_recv()
            if h < N_HOP - 1:
                rdma(of_r, fwd_send, fwd_recv, h + 1, right).start()
            rdma(ob_r, bwd_send, bwd_recv, h, left).wait_recv()
            if h < N_HOP - 1:
                rdma(ob_r, bwd_send, bwd_recv, h + 1, left).start()

            gemm(allx_ref[of_r], of_r * M_PER)
            gemm(allx_ref[ob_r], ob_r * M_PER)

            if h == 0:
                rdma(anti, z_send, z_recv, 0, anti).wait_recv()
                gemm(allx_ref[anti], anti * M_PER)

        for h in range(N_HOP):
            rdma((my + N_DEV - h) % N_DEV, fwd_send, fwd_recv, h,
                 right).wait_send()
            rdma((my + h) % N_DEV, bwd_send, bwd_recv, h, left).wait_send()
        rdma(my, z_send, z_recv, 0, anti).wait_send()

    return pl.pallas_call(
        body,
        out_shape=jax.ShapeDtypeStruct((N_DEV * m_per, n_per), jnp.float32),
        in_specs=[
            pl.BlockSpec(memory_space=pltpu.VMEM),
            pl.BlockSpec(memory_space=pltpu.VMEM),
            pl.BlockSpec(memory_space=pltpu.SMEM),
        ],
        out_specs=pl.BlockSpec(memory_space=pltpu.VMEM),
        scratch_shapes=[
            pltpu.VMEM((N_DEV, M_PER, k), jnp.int8),
            pltpu.SemaphoreType.DMA((N_HOP,)),
            pltpu.SemaphoreType.DMA((N_HOP,)),
            pltpu.SemaphoreType.DMA((N_HOP,)),
            pltpu.SemaphoreType.DMA((N_HOP,)),
            pltpu.SemaphoreType.DMA((1,)),
            pltpu.SemaphoreType.DMA((1,)),
        ],
        compiler_params=pltpu.CompilerParams(
            collective_id=0, vmem_limit_bytes=100 * 1024 * 1024,
        ),
    )(x, w_mat, scale)
